# baseline (speedup 1.0000x reference)
import functools

import jax
import jax.numpy as jnp
from jax import lax
from jax.experimental import pallas as pl
from jax.experimental.pallas import tpu as pltpu
from jax.experimental.pallas import tpu_sc as plsc

N = 16384
D = 64
NC = 2
NS = 16
NW = NC * NS
CHUNK = N // NW
ROWS = CHUNK * 3
GCHUNK = 128
NG = ROWS // GCHUNK
L = 16


def _build_kernel(off_dev: int, off_sex: int):
    mesh = plsc.VectorSubcoreMesh(core_axis_name="c", subcore_axis_name="s")

    @functools.partial(
        pl.kernel,
        mesh=mesh,
        out_type=jax.ShapeDtypeStruct((N * 3, D), jnp.float32),
        compiler_params=pltpu.CompilerParams(needs_layout_passes=False,
                                             use_tc_tiling_on_sc=False),
        scratch_types=[
            pltpu.VMEM((CHUNK,), jnp.int32),
            pltpu.VMEM((CHUNK,), jnp.int32),
            pltpu.VMEM((CHUNK,), jnp.int32),
            pltpu.VMEM((ROWS,), jnp.int32),
            pltpu.VMEM((ROWS, D), jnp.float32),
            pltpu.SemaphoreType.DMA,
        ],
    )
    def body(cell_hbm, dev_hbm, sex_hbm, table_hbm, out_hbm,
             idx_c, idx_d, idx_s, comb, rows, sem):
        wid = lax.axis_index("s") * NC + lax.axis_index("c")
        base = wid * CHUNK
        pltpu.sync_copy(cell_hbm.at[pl.ds(base, CHUNK)], idx_c)
        pltpu.sync_copy(dev_hbm.at[pl.ds(base, CHUNK)], idx_d)
        pltpu.sync_copy(sex_hbm.at[pl.ds(base, CHUNK)], idx_s)

        pos0 = lax.iota(jnp.int32, L) * 3
        for i in range(CHUNK // L):
            pos = pos0 + (3 * L * i)
            c = idx_c[pl.ds(i * L, L)]
            d = idx_d[pl.ds(i * L, L)] + off_dev
            s = idx_s[pl.ds(i * L, L)] + off_sex
            plsc.store_scatter(comb, [pos], c)
            plsc.store_scatter(comb, [pos + 1], d)
            plsc.store_scatter(comb, [pos + 2], s)

        copies = []
        for j in range(NG):
            copies.append(pltpu.async_copy(
                table_hbm.at[comb.at[pl.ds(j * GCHUNK, GCHUNK)]],
                rows.at[pl.ds(j * GCHUNK, GCHUNK)],
                sem,
            ))
        for cp in copies:
            cp.wait()

        pltpu.sync_copy(rows, out_hbm.at[pl.ds(base * 3, ROWS)])

    return body


def kernel(cell_type, development_stage, sex,
           E_cell_type, E_development_stage, E_sex):
    n_cell = E_cell_type.shape[0]
    n_dev = E_development_stage.shape[0]
    table = jnp.concatenate([E_cell_type, E_development_stage, E_sex], axis=0)
    body = _build_kernel(n_cell, n_cell + n_dev)
    out = body(cell_type.astype(jnp.int32),
               development_stage.astype(jnp.int32),
               sex.astype(jnp.int32),
               table)
    return out.reshape(N, 3, D)


# speedup vs baseline: 1.4895x; 1.4895x over previous
import functools

import jax
import jax.numpy as jnp
from jax import lax
from jax.experimental import pallas as pl
from jax.experimental.pallas import tpu as pltpu
from jax.experimental.pallas import tpu_sc as plsc

N = 16384
D = 64
NC = 2
NS = 16
NW = NC * NS
CHUNK = N // NW
Q = 128
NQ = CHUNK // Q
GPQ = Q // 16
L = 16


def _build_kernel(n_cell: int, n_dev: int, n_sex: int):
    mesh = plsc.VectorSubcoreMesh(core_axis_name="c", subcore_axis_name="s")

    @functools.partial(
        pl.kernel,
        mesh=mesh,
        out_type=jax.ShapeDtypeStruct((N * 3 * D,), jnp.float32),
        compiler_params=pltpu.CompilerParams(needs_layout_passes=False,
                                             use_tc_tiling_on_sc=False),
        scratch_types=[
            pltpu.VMEM((CHUNK,), jnp.int32),
            pltpu.VMEM((CHUNK,), jnp.int32),
            pltpu.VMEM((CHUNK,), jnp.int32),
            pltpu.VMEM((n_cell * D,), jnp.float32),
            pltpu.VMEM((n_dev * D,), jnp.float32),
            pltpu.VMEM((n_sex * D,), jnp.float32),
            pltpu.VMEM((Q * 3 * D,), jnp.float32),
            pltpu.VMEM((Q * 3 * D,), jnp.float32),
            pltpu.SemaphoreType.DMA,
            pltpu.SemaphoreType.DMA,
        ],
    )
    def body(cell_hbm, dev_hbm, sex_hbm, tc_hbm, td_hbm, ts_hbm, out_hbm,
             idx_c, idx_d, idx_s, tabc, tabd, tabs, obuf0, obuf1, sem0, sem1):
        wid = lax.axis_index("s") * NC + lax.axis_index("c")
        base = wid * CHUNK
        pltpu.sync_copy(tc_hbm, tabc)
        pltpu.sync_copy(td_hbm, tabd)
        pltpu.sync_copy(ts_hbm, tabs)
        pltpu.sync_copy(cell_hbm.at[pl.ds(base, CHUNK)], idx_c)
        pltpu.sync_copy(dev_hbm.at[pl.ds(base, CHUNK)], idx_d)
        pltpu.sync_copy(sex_hbm.at[pl.ds(base, CHUNK)], idx_s)

        iota = lax.iota(jnp.int32, L)
        pos_lane = iota * (3 * D)

        obufs = (obuf0, obuf1)
        sems = (sem0, sem1)
        pending = [None, None]

        def make_group(h, buf):
            def group(g, carry):
                t0 = h * Q + g * L
                rows_c = idx_c[pl.ds(t0, L)]
                rows_d = idx_d[pl.ds(t0, L)]
                rows_s = idx_s[pl.ds(t0, L)]
                posg = pos_lane + g * (L * 3 * D)
                for f, (tab, rows) in enumerate(
                        ((tabc, rows_c), (tabd, rows_d), (tabs, rows_s))):
                    rowbase = rows * D
                    posf = posg + f * D
                    for c in range(D):
                        val = plsc.load_gather(tab, [rowbase + c])
                        plsc.store_scatter(buf, [posf + c], val)
                return carry
            return group

        for h in range(NQ):
            b = h % 2
            if pending[b] is not None:
                pending[b].wait()
            lax.fori_loop(0, GPQ, make_group(h, obufs[b]), 0)
            pending[b] = pltpu.async_copy(
                obufs[b],
                out_hbm.at[pl.ds((base + h * Q) * 3 * D, Q * 3 * D)],
                sems[b],
            )
        for b in range(2):
            if pending[b] is not None:
                pending[b].wait()

    return body


def kernel(cell_type, development_stage, sex,
           E_cell_type, E_development_stage, E_sex):
    n_cell = E_cell_type.shape[0]
    n_dev = E_development_stage.shape[0]
    n_sex = E_sex.shape[0]
    body = _build_kernel(n_cell, n_dev, n_sex)
    out = body(cell_type.astype(jnp.int32),
               development_stage.astype(jnp.int32),
               sex.astype(jnp.int32),
               E_cell_type.reshape(-1),
               E_development_stage.reshape(-1),
               E_sex.reshape(-1))
    return out.reshape(N, 3, D)


# speedup vs baseline: 2.2119x; 1.4850x over previous
import functools

import jax
import jax.numpy as jnp
from jax import lax
from jax.experimental import pallas as pl
from jax.experimental.pallas import tpu as pltpu
from jax.experimental.pallas import tpu_sc as plsc

N = 16384
D = 64
NC = 2
NS = 16
NW = NC * NS
CHUNK = N // NW
Q = 128
NQ = CHUNK // Q
GPQ = Q // 16
L = 16


def _build_kernel(n_cell: int, n_dev: int, n_sex: int):
    mesh = plsc.VectorSubcoreMesh(core_axis_name="c", subcore_axis_name="s")

    @functools.partial(
        pl.kernel,
        mesh=mesh,
        out_type=jax.ShapeDtypeStruct((N * 3 * D,), jnp.float32),
        compiler_params=pltpu.CompilerParams(needs_layout_passes=False,
                                             use_tc_tiling_on_sc=False),
        scratch_types=[
            pltpu.VMEM((CHUNK,), jnp.int32),
            pltpu.VMEM((CHUNK,), jnp.int32),
            pltpu.VMEM((CHUNK,), jnp.int32),
            pltpu.VMEM((n_cell * D,), jnp.float32),
            pltpu.VMEM((n_dev * D,), jnp.float32),
            pltpu.VMEM((n_sex * D,), jnp.float32),
            pltpu.VMEM((Q * 3 * D,), jnp.float32),
            pltpu.VMEM((Q * 3 * D,), jnp.float32),
            pltpu.SemaphoreType.DMA,
            pltpu.SemaphoreType.DMA,
        ],
    )
    def body(cell_hbm, dev_hbm, sex_hbm, tc_hbm, td_hbm, ts_hbm, out_hbm,
             idx_c, idx_d, idx_s, tabc, tabd, tabs, obuf0, obuf1, sem0, sem1):
        wid = lax.axis_index("s") * NC + lax.axis_index("c")
        base = wid * CHUNK
        pltpu.sync_copy(tc_hbm, tabc)
        pltpu.sync_copy(td_hbm, tabd)
        pltpu.sync_copy(ts_hbm, tabs)
        pltpu.sync_copy(cell_hbm.at[pl.ds(base, CHUNK)], idx_c)
        pltpu.sync_copy(dev_hbm.at[pl.ds(base, CHUNK)], idx_d)
        pltpu.sync_copy(sex_hbm.at[pl.ds(base, CHUNK)], idx_s)

        iota = lax.iota(jnp.int32, L)
        pos_lane = iota * (3 * D)

        obufs = (obuf0, obuf1)
        sems = (sem0, sem1)
        pending = [None, None]

        GROUP_ELEMS = L * 3 * D
        CB = D // L

        def make_field(h, buf, f, tab, idx_ref):
            def field_blk(e):
                g = e >> 2
                k = e & 3
                t0 = h * Q + g * L
                rows = idx_ref[pl.ds(t0, L)]
                rowbase = rows * D + k * L
                gbuf = buf.at[pl.ds(g * GROUP_ELEMS, GROUP_ELEMS)]
                posk = pos_lane + (f * D) + k * L
                for c in range(L):
                    val = plsc.load_gather(tab, [rowbase + c])
                    plsc.store_scatter(gbuf, [posk + c], val)
            return field_blk

        for h in range(NQ):
            b = h % 2
            if pending[b] is not None:
                pending[b].wait()
            for f, (tab, idx_ref) in enumerate(
                    ((tabc, idx_c), (tabd, idx_d), (tabs, idx_s))):
                plsc.parallel_loop(0, GPQ * CB)(
                    make_field(h, obufs[b], f, tab, idx_ref))
            pending[b] = pltpu.async_copy(
                obufs[b],
                out_hbm.at[pl.ds((base + h * Q) * 3 * D, Q * 3 * D)],
                sems[b],
            )
        for b in range(2):
            if pending[b] is not None:
                pending[b].wait()

    return body


def kernel(cell_type, development_stage, sex,
           E_cell_type, E_development_stage, E_sex):
    n_cell = E_cell_type.shape[0]
    n_dev = E_development_stage.shape[0]
    n_sex = E_sex.shape[0]
    body = _build_kernel(n_cell, n_dev, n_sex)
    out = body(cell_type.astype(jnp.int32),
               development_stage.astype(jnp.int32),
               sex.astype(jnp.int32),
               E_cell_type.reshape(-1),
               E_development_stage.reshape(-1),
               E_sex.reshape(-1))
    return out.reshape(N, 3, D)


# speedup vs baseline: 2.8788x; 1.3015x over previous
import functools

import jax
import jax.numpy as jnp
from jax import lax
from jax.experimental import pallas as pl
from jax.experimental.pallas import tpu as pltpu
from jax.experimental.pallas import tpu_sc as plsc

N = 16384
D = 64
NC = 2
NS = 16
NW = NC * NS
CHUNK = N // NW
Q = 128
NQ = CHUNK // Q
L = 16
CB = D // L


def _build_kernel(n_cell: int, n_dev: int, n_sex: int):
    mesh = plsc.VectorSubcoreMesh(core_axis_name="c", subcore_axis_name="s")

    @functools.partial(
        pl.kernel,
        mesh=mesh,
        out_type=jax.ShapeDtypeStruct((N * 3 * D,), jnp.float32),
        compiler_params=pltpu.CompilerParams(needs_layout_passes=False,
                                             use_tc_tiling_on_sc=False),
        scratch_types=[
            pltpu.VMEM((CHUNK,), jnp.int32),
            pltpu.VMEM((CHUNK,), jnp.int32),
            pltpu.VMEM((CHUNK,), jnp.int32),
            pltpu.VMEM((n_cell * D,), jnp.float32),
            pltpu.VMEM((n_dev * D,), jnp.float32),
            pltpu.VMEM((n_sex * D,), jnp.float32),
            pltpu.VMEM((Q * 3 * D,), jnp.float32),
            pltpu.VMEM((Q * 3 * D,), jnp.float32),
            pltpu.SemaphoreType.DMA,
            pltpu.SemaphoreType.DMA,
        ],
    )
    def body(cell_hbm, dev_hbm, sex_hbm, tc_hbm, td_hbm, ts_hbm, out_hbm,
             idx_c, idx_d, idx_s, tabc, tabd, tabs, obuf0, obuf1, sem0, sem1):
        wid = lax.axis_index("s") * NC + lax.axis_index("c")
        base = wid * CHUNK
        staging = [
            pltpu.async_copy(tc_hbm, tabc, sem0),
            pltpu.async_copy(td_hbm, tabd, sem0),
            pltpu.async_copy(ts_hbm, tabs, sem0),
            pltpu.async_copy(cell_hbm.at[pl.ds(base, CHUNK)], idx_c, sem0),
            pltpu.async_copy(dev_hbm.at[pl.ds(base, CHUNK)], idx_d, sem0),
            pltpu.async_copy(sex_hbm.at[pl.ds(base, CHUNK)], idx_s, sem0),
        ]
        for cp in staging:
            cp.wait()

        obufs = (obuf0, obuf1)
        sems = (sem0, sem1)
        pending = [None, None]

        def make_slab(h, buf):
            def group(g):
                t0 = h * Q + g * L
                rows_c = idx_c[pl.ds(t0, L)]
                rows_d = idx_d[pl.ds(t0, L)]
                rows_s = idx_s[pl.ds(t0, L)]
                gbase = g * (L * 3 * D)
                for l in range(L):
                    ot = gbase + l * (3 * D)
                    for f, (tab, rows) in enumerate(
                            ((tabc, rows_c), (tabd, rows_d), (tabs, rows_s))):
                        rb = rows[l] * D
                        for k in range(CB):
                            val = tab[pl.ds(rb + k * L, L)]
                            buf[pl.ds(ot + f * D + k * L, L)] = val
            return group

        for h in range(NQ):
            b = h % 2
            if pending[b] is not None:
                pending[b].wait()
            plsc.parallel_loop(0, Q // L)(make_slab(h, obufs[b]))
            pending[b] = pltpu.async_copy(
                obufs[b],
                out_hbm.at[pl.ds((base + h * Q) * 3 * D, Q * 3 * D)],
                sems[b],
            )
        for b in range(2):
            if pending[b] is not None:
                pending[b].wait()

    return body


def kernel(cell_type, development_stage, sex,
           E_cell_type, E_development_stage, E_sex):
    n_cell = E_cell_type.shape[0]
    n_dev = E_development_stage.shape[0]
    n_sex = E_sex.shape[0]
    body = _build_kernel(n_cell, n_dev, n_sex)
    out = body(cell_type.astype(jnp.int32),
               development_stage.astype(jnp.int32),
               sex.astype(jnp.int32),
               E_cell_type.reshape(-1),
               E_development_stage.reshape(-1),
               E_sex.reshape(-1))
    return out.reshape(N, 3, D)


# speedup vs baseline: 4.4230x; 1.5364x over previous
import functools

import jax
import jax.numpy as jnp
from jax import lax
from jax.experimental import pallas as pl
from jax.experimental.pallas import tpu as pltpu
from jax.experimental.pallas import tpu_sc as plsc

N = 16384
D = 64
NC = 2
NS = 16
NW = NC * NS
CHUNK = N // NW
Q = 128
NQ = CHUNK // Q
L = 16
GPQ = Q // L
RS = D + 1


def _build_kernel(n_cell: int, n_dev: int, n_sex: int):
    mesh = plsc.VectorSubcoreMesh(core_axis_name="c", subcore_axis_name="s")

    @functools.partial(
        pl.kernel,
        mesh=mesh,
        out_type=jax.ShapeDtypeStruct((3 * D, N), jnp.float32),
        compiler_params=pltpu.CompilerParams(needs_layout_passes=False,
                                             use_tc_tiling_on_sc=False),
        scratch_types=[
            pltpu.VMEM((CHUNK,), jnp.int32),
            pltpu.VMEM((CHUNK,), jnp.int32),
            pltpu.VMEM((CHUNK,), jnp.int32),
            pltpu.VMEM((n_cell * RS + D,), jnp.float32),
            pltpu.VMEM((n_dev * RS + D,), jnp.float32),
            pltpu.VMEM((n_sex * L * RS + D,), jnp.float32),
            pltpu.VMEM((3 * D, Q), jnp.float32),
            pltpu.SemaphoreType.DMA,
            pltpu.SemaphoreType.DMA,
        ],
    )
    def body(cell_hbm, dev_hbm, sex_hbm, tc_hbm, td_hbm, ts_hbm, out_hbm,
             idx_c, idx_d, idx_s, tabc, tabd, tabs, obuf0, sem0, sem1):
        wid = lax.axis_index("s") * NC + lax.axis_index("c")
        base = wid * CHUNK
        staging = [
            pltpu.async_copy(tc_hbm, tabc.at[pl.ds(0, n_cell * RS)], sem0),
            pltpu.async_copy(td_hbm, tabd.at[pl.ds(0, n_dev * RS)], sem0),
            pltpu.async_copy(ts_hbm, tabs.at[pl.ds(0, n_sex * L * RS)], sem0),
            pltpu.async_copy(cell_hbm.at[pl.ds(base, CHUNK)], idx_c, sem0),
            pltpu.async_copy(dev_hbm.at[pl.ds(base, CHUNK)], idx_d, sem0),
            pltpu.async_copy(sex_hbm.at[pl.ds(base, CHUNK)], idx_s, sem0),
        ]
        for cp in staging:
            cp.wait()

        iota = lax.iota(jnp.int32, L)

        KB = 8
        tabc_k = [tabc.at[pl.ds(k * KB, n_cell * RS)] for k in range(D // KB)]
        tabd_k = [tabd.at[pl.ds(k * KB, n_dev * RS)] for k in range(D // KB)]
        tabs_k = [tabs.at[pl.ds(k * KB, n_sex * L * RS)]
                  for k in range(D // KB)]

        def make_slab(h, buf):
            def group(g):
                t0 = h * Q + g * L
                c0 = g * L
                gb_c = idx_c[pl.ds(t0, L)] * RS
                gb_d = idx_d[pl.ds(t0, L)] * RS
                gb_s = (idx_s[pl.ds(t0, L)] * L + iota) * RS
                for f, (tab_k, gbase) in enumerate(
                        ((tabc_k, gb_c), (tabd_k, gb_d), (tabs_k, gb_s))):
                    gvecs = [gbase + cp for cp in range(KB)]
                    for k in range(D // KB):
                        for cp in range(KB):
                            val = plsc.load_gather(tab_k[k], [gvecs[cp]])
                            buf[f * D + k * KB + cp, pl.ds(c0, L)] = val
            return group

        def slab(h, carry):
            plsc.parallel_loop(0, GPQ)(make_slab(h, obuf0))
            pltpu.sync_copy(obuf0, out_hbm.at[:, pl.ds(base + h * Q, Q)])
            return carry

        lax.fori_loop(0, NQ, slab, 0)

    return body


def kernel(cell_type, development_stage, sex,
           E_cell_type, E_development_stage, E_sex):
    n_cell = E_cell_type.shape[0]
    n_dev = E_development_stage.shape[0]
    n_sex = E_sex.shape[0]
    pad = [(0, 0), (0, RS - D)]
    tabc = jnp.pad(E_cell_type, pad).reshape(-1)
    tabd = jnp.pad(E_development_stage, pad).reshape(-1)
    tabs = jnp.pad(jnp.repeat(E_sex, L, axis=0), pad).reshape(-1)
    body = _build_kernel(n_cell, n_dev, n_sex)
    out = body(cell_type.astype(jnp.int32),
               development_stage.astype(jnp.int32),
               sex.astype(jnp.int32),
               tabc, tabd, tabs)
    return out.reshape(3, D, N).transpose(2, 0, 1)


# speedup vs baseline: 5.8291x; 1.3179x over previous
import functools

import jax
import jax.numpy as jnp
from jax import lax
from jax.experimental import pallas as pl
from jax.experimental.pallas import tpu as pltpu
from jax.experimental.pallas import tpu_sc as plsc

N = 16384
D = 64
NC = 2
NS = 16
NW = NC * NS
CHUNK = N // NW
Q = 128
NQ = CHUNK // Q
L = 16
GPQ = Q // L
RS = D + 1


def _build_kernel(n_cell: int, n_dev: int, n_sex: int):
    mesh = plsc.VectorSubcoreMesh(core_axis_name="c", subcore_axis_name="s")

    @functools.partial(
        pl.kernel,
        mesh=mesh,
        out_type=jax.ShapeDtypeStruct((3, D // 8, N // Q, 8 * Q), jnp.float32),
        compiler_params=pltpu.CompilerParams(needs_layout_passes=False,
                                             use_tc_tiling_on_sc=False),
        scratch_types=[
            pltpu.VMEM((CHUNK,), jnp.int32),
            pltpu.VMEM((CHUNK,), jnp.int32),
            pltpu.VMEM((CHUNK,), jnp.int32),
            pltpu.VMEM((n_cell * RS + D,), jnp.float32),
            pltpu.VMEM((n_dev * RS + D,), jnp.float32),
            pltpu.VMEM((n_sex * L * RS + D,), jnp.float32),
            pltpu.VMEM((3, D // 8, 1, 8 * Q), jnp.float32),
            pltpu.SemaphoreType.DMA,
            pltpu.SemaphoreType.DMA,
        ],
    )
    def body(cell_hbm, dev_hbm, sex_hbm, tc_hbm, td_hbm, ts_hbm, out_hbm,
             idx_c, idx_d, idx_s, tabc, tabd, tabs, obuf0, sem0, sem1):
        wid = lax.axis_index("s") * NC + lax.axis_index("c")
        base = wid * CHUNK
        staging = [
            pltpu.async_copy(tc_hbm, tabc.at[pl.ds(0, n_cell * RS)], sem0),
            pltpu.async_copy(td_hbm, tabd.at[pl.ds(0, n_dev * RS)], sem0),
            pltpu.async_copy(ts_hbm, tabs.at[pl.ds(0, n_sex * L * RS)], sem0),
            pltpu.async_copy(cell_hbm.at[pl.ds(base, CHUNK)], idx_c, sem0),
            pltpu.async_copy(dev_hbm.at[pl.ds(base, CHUNK)], idx_d, sem0),
            pltpu.async_copy(sex_hbm.at[pl.ds(base, CHUNK)], idx_s, sem0),
        ]
        for cp in staging:
            cp.wait()

        iota = lax.iota(jnp.int32, L)

        KB = 8
        tabc_k = [tabc.at[pl.ds(k * KB, n_cell * RS)] for k in range(D // KB)]
        tabd_k = [tabd.at[pl.ds(k * KB, n_dev * RS)] for k in range(D // KB)]
        tabs_k = [tabs.at[pl.ds(k * KB, n_sex * L * RS)]
                  for k in range(D // KB)]

        def make_slab(h, buf):
            def group(g):
                t0 = h * Q + g * L
                c0 = g * L
                gb_c = idx_c[pl.ds(t0, L)] * RS
                gb_d = idx_d[pl.ds(t0, L)] * RS
                gb_s = (idx_s[pl.ds(t0, L)] * L + iota) * RS
                for f, (tab_k, gbase) in enumerate(
                        ((tabc_k, gb_c), (tabd_k, gb_d), (tabs_k, gb_s))):
                    gvecs = [gbase + cp for cp in range(KB)]
                    for k in range(D // KB):
                        for cp in range(KB):
                            val = plsc.load_gather(tab_k[k], [gvecs[cp]])
                            buf[f, k, 0, pl.ds(cp * Q + c0, L)] = val
            return group

        def slab(h, carry):
            plsc.parallel_loop(0, GPQ)(make_slab(h, obuf0))
            pltpu.sync_copy(
                obuf0, out_hbm.at[:, :, pl.ds(wid * NQ + h, 1), :])
            return carry

        lax.fori_loop(0, NQ, slab, 0)

    return body


def kernel(cell_type, development_stage, sex,
           E_cell_type, E_development_stage, E_sex):
    n_cell = E_cell_type.shape[0]
    n_dev = E_development_stage.shape[0]
    n_sex = E_sex.shape[0]
    pad = [(0, 0), (0, RS - D)]
    tabc = jnp.pad(E_cell_type, pad).reshape(-1)
    tabd = jnp.pad(E_development_stage, pad).reshape(-1)
    tabs = jnp.pad(jnp.repeat(E_sex, L, axis=0), pad).reshape(-1)
    body = _build_kernel(n_cell, n_dev, n_sex)
    out = body(cell_type.astype(jnp.int32),
               development_stage.astype(jnp.int32),
               sex.astype(jnp.int32),
               tabc, tabd, tabs)
    out = out.reshape(3, D // 8, N // Q, 8, Q).transpose(0, 1, 3, 2, 4)
    return out.reshape(3, D, N).transpose(2, 0, 1)


# speedup vs baseline: 6.5787x; 1.1286x over previous
import functools

import jax
import jax.numpy as jnp
from jax import lax
from jax.experimental import pallas as pl
from jax.experimental.pallas import tpu as pltpu
from jax.experimental.pallas import tpu_sc as plsc

N = 16384
D = 64
NC = 2
NS = 16
NW = NC * NS
CHUNK = N // NW
Q = 128
NQ = CHUNK // Q
L = 16
GPQ = Q // L
RS = D + 1


def _build_kernel(n_cell: int, n_dev: int, n_sex: int):
    mesh = plsc.VectorSubcoreMesh(core_axis_name="c", subcore_axis_name="s")

    @functools.partial(
        pl.kernel,
        mesh=mesh,
        out_type=jax.ShapeDtypeStruct((3, D // 8, N // Q, 8 * Q), jnp.float32),
        compiler_params=pltpu.CompilerParams(needs_layout_passes=False,
                                             use_tc_tiling_on_sc=False),
        scratch_types=[
            pltpu.VMEM((CHUNK,), jnp.int32),
            pltpu.VMEM((CHUNK,), jnp.int32),
            pltpu.VMEM((CHUNK,), jnp.int32),
            pltpu.VMEM((n_cell * RS + D,), jnp.float32),
            pltpu.VMEM((n_dev * RS + D,), jnp.float32),
            pltpu.VMEM((n_sex * L * RS + D,), jnp.float32),
            pltpu.VMEM((1, D // 8, 1, 8 * Q), jnp.float32),
            pltpu.VMEM((1, D // 8, 1, 8 * Q), jnp.float32),
            pltpu.VMEM((1, D // 8, 1, 8 * Q), jnp.float32),
            pltpu.SemaphoreType.DMA,
            pltpu.SemaphoreType.DMA,
            pltpu.SemaphoreType.DMA,
            pltpu.SemaphoreType.DMA,
        ],
    )
    def body(cell_hbm, dev_hbm, sex_hbm, tc_hbm, td_hbm, ts_hbm, out_hbm,
             idx_c, idx_d, idx_s, tabc, tabd, tabs,
             fbuf0, fbuf1, fbuf2, sem0, fsem0, fsem1, fsem2):
        wid = lax.axis_index("s") * NC + lax.axis_index("c")
        base = wid * CHUNK
        staging = [
            pltpu.async_copy(tc_hbm, tabc.at[pl.ds(0, n_cell * RS)], sem0),
            pltpu.async_copy(td_hbm, tabd.at[pl.ds(0, n_dev * RS)], sem0),
            pltpu.async_copy(ts_hbm, tabs.at[pl.ds(0, n_sex * L * RS)], sem0),
            pltpu.async_copy(cell_hbm.at[pl.ds(base, CHUNK)], idx_c, sem0),
            pltpu.async_copy(dev_hbm.at[pl.ds(base, CHUNK)], idx_d, sem0),
            pltpu.async_copy(sex_hbm.at[pl.ds(base, CHUNK)], idx_s, sem0),
        ]
        for cp in staging:
            cp.wait()

        iota = lax.iota(jnp.int32, L)

        KB = 8
        tabc_k = [tabc.at[pl.ds(k * KB, n_cell * RS)] for k in range(D // KB)]
        tabd_k = [tabd.at[pl.ds(k * KB, n_dev * RS)] for k in range(D // KB)]
        tabs_k = [tabs.at[pl.ds(k * KB, n_sex * L * RS)]
                  for k in range(D // KB)]

        def make_field(h, buf, tab_k, idx_ref, is_sex):
            def group(g):
                t0 = h * Q + g * L
                c0 = g * L
                rows = idx_ref[pl.ds(t0, L)]
                gbase = (rows * L + iota) * RS if is_sex else rows * RS
                gvecs = [gbase + cp for cp in range(KB)]
                for k in range(D // KB):
                    for cp in range(KB):
                        val = plsc.load_gather(tab_k[k], [gvecs[cp]])
                        buf[0, k, 0, pl.ds(cp * Q + c0, L)] = val
            return group

        fbufs = (fbuf0, fbuf1, fbuf2)
        fsems = (fsem0, fsem1, fsem2)
        fields = ((tabc_k, idx_c, False),
                  (tabd_k, idx_d, False),
                  (tabs_k, idx_s, True))

        def slab(h, carry):
            for f in range(3):
                tab_k, idx_ref, is_sex = fields[f]
                buf, sem = fbufs[f], fsems[f]

                @pl.when(h > 0)
                def _wait_prev(buf=buf, sem=sem, f=f):
                    pltpu.make_async_copy(
                        buf,
                        out_hbm.at[pl.ds(f, 1), :,
                                   pl.ds(wid * NQ + h - 1, 1), :],
                        sem).wait()

                plsc.parallel_loop(0, GPQ)(
                    make_field(h, buf, tab_k, idx_ref, is_sex))
                pltpu.async_copy(
                    buf,
                    out_hbm.at[pl.ds(f, 1), :, pl.ds(wid * NQ + h, 1), :],
                    sem)
            return carry

        lax.fori_loop(0, NQ, slab, 0)
        for f in range(3):
            pltpu.make_async_copy(
                fbufs[f],
                out_hbm.at[pl.ds(f, 1), :, pl.ds(wid * NQ + NQ - 1, 1), :],
                fsems[f]).wait()

    return body


def kernel(cell_type, development_stage, sex,
           E_cell_type, E_development_stage, E_sex):
    n_cell = E_cell_type.shape[0]
    n_dev = E_development_stage.shape[0]
    n_sex = E_sex.shape[0]
    pad = [(0, 0), (0, RS - D)]
    tabc = jnp.pad(E_cell_type, pad).reshape(-1)
    tabd = jnp.pad(E_development_stage, pad).reshape(-1)
    tabs = jnp.pad(jnp.repeat(E_sex, L, axis=0), pad).reshape(-1)
    body = _build_kernel(n_cell, n_dev, n_sex)
    out = body(cell_type.astype(jnp.int32),
               development_stage.astype(jnp.int32),
               sex.astype(jnp.int32),
               tabc, tabd, tabs)
    out = out.reshape(3, D // 8, N // Q, 8, Q).transpose(0, 1, 3, 2, 4)
    return out.reshape(3, D, N).transpose(2, 0, 1)
